# Initial kernel scaffold; baseline (speedup 1.0000x reference)
import jax
import jax.numpy as jnp
from jax.experimental import pallas as pl


def kernel(x, adj, W1, b1, W2, b2, fc1_w, fc1_b, fc2_w, fc2_b):
    raise NotImplementedError("write your pallas kernel here")


# speedup vs baseline: 1.5835x; 1.5835x over previous
import jax
import jax.numpy as jnp
from jax.experimental import pallas as pl
from jax.experimental.pallas import tpu as pltpu

N = 208
NFEAT = 512
NHID = 256
NCLASS = 128


def _fused(x_ref, adj_ref, w1_ref, b1_ref, w2_ref, b2_ref,
           fc1w_ref, fc1b_ref, fc2w_ref, fc2b_ref, out_ref):
    x = x_ref[...]
    adj = adj_ref[...]
    t1 = jnp.dot(x, w1_ref[...], preferred_element_type=jnp.float32)
    h1 = jnp.maximum(jnp.dot(adj, t1, preferred_element_type=jnp.float32)
                     + b1_ref[...], 0.0)
    t2 = jnp.dot(h1, w2_ref[...], preferred_element_type=jnp.float32)
    h2 = jnp.maximum(jnp.dot(adj, t2, preferred_element_type=jnp.float32)
                     + b2_ref[...], 0.0)
    flat = h2.reshape(1, N * NCLASS)
    h3 = jax.lax.dot_general(flat, fc1w_ref[...],
                             (((1,), (1,)), ((), ())),
                             preferred_element_type=jnp.float32)
    h3 = jnp.maximum(h3 + fc1b_ref[...], 0.0)
    z = jnp.sum(h3 * fc2w_ref[...], axis=1, keepdims=True)
    out_ref[...] = jax.nn.sigmoid(z + fc2b_ref[0, 0])


def kernel(x, adj, W1, b1, W2, b2, fc1_w, fc1_b, fc2_w, fc2_b):
    out = pl.pallas_call(
        _fused,
        out_shape=jax.ShapeDtypeStruct((1, 1), jnp.float32),
        in_specs=[pl.BlockSpec(memory_space=pltpu.VMEM)] * 9
                 + [pl.BlockSpec(memory_space=pltpu.SMEM)],
        out_specs=pl.BlockSpec(memory_space=pltpu.VMEM),
    )(x, adj, W1, b1.reshape(1, NHID), W2, b2.reshape(1, NCLASS),
      fc1_w, fc1_b.reshape(1, 60), fc2_w, fc2_b.reshape(1, 1))
    return out.reshape(1)
